# baseline (device time: 8504 ns/iter reference)
import jax
import jax.numpy as jnp
from jax import lax
from jax.experimental import pallas as pl
from jax.experimental.pallas import tpu as pltpu

K = 8
NEG_INF = float("-inf")


def _topk_iterative(vals, k):
    outs = []
    cur = vals
    for _ in range(k):
        m = jnp.max(cur, axis=1, keepdims=True)
        outs.append(m)
        cur = jnp.where(cur == m, NEG_INF, cur)
    return jnp.concatenate(outs, axis=1)


def kernel(x):
    m, n = x.shape

    def body(x_ref, out_ref, cand_ref, send_sem, recv_sem):
        my_x = lax.axis_index("x")
        my_y = lax.axis_index("y")
        peer = (my_x, 1 - my_y)

        cand_ref[0, :, :] = _topk_iterative(x_ref[:, :], K)

        barrier_sem = pltpu.get_barrier_semaphore()
        pl.semaphore_signal(
            barrier_sem, inc=1, device_id=peer,
            device_id_type=pl.DeviceIdType.MESH,
        )
        pl.semaphore_wait(barrier_sem, 1)

        rdma = pltpu.make_async_remote_copy(
            src_ref=cand_ref.at[0],
            dst_ref=cand_ref.at[1],
            send_sem=send_sem,
            recv_sem=recv_sem,
            device_id=peer,
            device_id_type=pl.DeviceIdType.MESH,
        )
        rdma.start()
        rdma.wait()

        merged = jnp.concatenate([cand_ref[0, :, :], cand_ref[1, :, :]], axis=1)
        out_ref[:, :] = _topk_iterative(merged, K)

    return pl.pallas_call(
        body,
        out_shape=jax.ShapeDtypeStruct((m, K), jnp.float32),
        in_specs=[pl.BlockSpec(memory_space=pltpu.VMEM)],
        out_specs=pl.BlockSpec(memory_space=pltpu.VMEM),
        scratch_shapes=[
            pltpu.VMEM((2, m, K), jnp.float32),
            pltpu.SemaphoreType.DMA,
            pltpu.SemaphoreType.DMA,
        ],
        compiler_params=pltpu.CompilerParams(collective_id=0),
    )(x)


# device time: 6656 ns/iter; 1.2776x vs baseline; 1.2776x over previous
import jax
import jax.numpy as jnp
from jax import lax
from jax.experimental import pallas as pl
from jax.experimental.pallas import tpu as pltpu

K = 8
NEG_INF = float("-inf")


def kernel(x):
    m, n = x.shape

    def body(x_ref, out_ref, cand_ref, send_sem, recv_sem):
        my_x = lax.axis_index("x")
        my_y = lax.axis_index("y")
        peer = (my_x, 1 - my_y)

        cur = x_ref[:, :]
        cols = []
        for _ in range(K):
            mx = jnp.max(cur, axis=1, keepdims=True)
            cols.append(mx)
            cur = jnp.where(cur == mx, NEG_INF, cur)
        local_top = jnp.concatenate(cols, axis=1)
        cand_ref[0, :, :] = local_top.T

        barrier_sem = pltpu.get_barrier_semaphore()
        pl.semaphore_signal(
            barrier_sem, inc=1, device_id=peer,
            device_id_type=pl.DeviceIdType.MESH,
        )
        pl.semaphore_wait(barrier_sem, 1)

        rdma = pltpu.make_async_remote_copy(
            src_ref=cand_ref.at[0],
            dst_ref=cand_ref.at[1],
            send_sem=send_sem,
            recv_sem=recv_sem,
            device_id=peer,
            device_id_type=pl.DeviceIdType.MESH,
        )
        rdma.start()
        rdma.wait()

        merged = jnp.concatenate([cand_ref[0, :, :], cand_ref[1, :, :]], axis=0)
        rows = []
        cur2 = merged
        for _ in range(K):
            mx = jnp.max(cur2, axis=0, keepdims=True)
            rows.append(mx)
            cur2 = jnp.where(cur2 == mx, NEG_INF, cur2)
        out_ref[:, :] = jnp.concatenate(rows, axis=0).T

    return pl.pallas_call(
        body,
        out_shape=jax.ShapeDtypeStruct((m, K), jnp.float32),
        in_specs=[pl.BlockSpec(memory_space=pltpu.VMEM)],
        out_specs=pl.BlockSpec(memory_space=pltpu.VMEM),
        scratch_shapes=[
            pltpu.VMEM((2, K, m), jnp.float32),
            pltpu.SemaphoreType.DMA,
            pltpu.SemaphoreType.DMA,
        ],
        compiler_params=pltpu.CompilerParams(collective_id=0),
    )(x)


# device time: 6574 ns/iter; 1.2936x vs baseline; 1.0125x over previous
import jax
import jax.numpy as jnp
from jax import lax
from jax.experimental import pallas as pl
from jax.experimental.pallas import tpu as pltpu

K = 8
NEG_INF = float("-inf")


def kernel(x):
    m, n = x.shape

    def body(x_ref, out_ref, cand_ref, send_sem, recv_sem):
        my_x = lax.axis_index("x")
        my_y = lax.axis_index("y")
        peer = (my_x, 1 - my_y)

        q = n // 4
        a = x_ref[:, 0 * q:1 * q]
        b = x_ref[:, 1 * q:2 * q]
        c = x_ref[:, 2 * q:3 * q]
        d = x_ref[:, 3 * q:4 * q]

        def ce(u, v):
            return jnp.maximum(u, v), jnp.minimum(u, v)

        p0, p1 = ce(a, b)
        p2, p3 = ce(c, d)
        s0, t2 = ce(p0, p2)
        t1, s3 = ce(p1, p3)
        s1, s2 = ce(t1, t2)

        cols = []
        for _ in range(K):
            mx = jnp.max(s0, axis=1, keepdims=True)
            cols.append(mx)
            hit = s0 == mx
            s0 = jnp.where(hit, s1, s0)
            s1 = jnp.where(hit, s2, s1)
            s2 = jnp.where(hit, s3, s2)
            s3 = jnp.where(hit, NEG_INF, s3)
        local_top = jnp.concatenate(cols, axis=1)
        cand_ref[0, :, :] = local_top.T

        barrier_sem = pltpu.get_barrier_semaphore()
        pl.semaphore_signal(
            barrier_sem, inc=1, device_id=peer,
            device_id_type=pl.DeviceIdType.MESH,
        )
        pl.semaphore_wait(barrier_sem, 1)

        rdma = pltpu.make_async_remote_copy(
            src_ref=cand_ref.at[0],
            dst_ref=cand_ref.at[1],
            send_sem=send_sem,
            recv_sem=recv_sem,
            device_id=peer,
            device_id_type=pl.DeviceIdType.MESH,
        )
        rdma.start()
        rdma.wait()

        merged = jnp.concatenate([cand_ref[0, :, :], cand_ref[1, :, :]], axis=0)
        rows = []
        cur2 = merged
        for _ in range(K):
            mx = jnp.max(cur2, axis=0, keepdims=True)
            rows.append(mx)
            cur2 = jnp.where(cur2 == mx, NEG_INF, cur2)
        out_ref[:, :] = jnp.concatenate(rows, axis=0).T

    return pl.pallas_call(
        body,
        out_shape=jax.ShapeDtypeStruct((m, K), jnp.float32),
        in_specs=[pl.BlockSpec(memory_space=pltpu.VMEM)],
        out_specs=pl.BlockSpec(memory_space=pltpu.VMEM),
        scratch_shapes=[
            pltpu.VMEM((2, K, m), jnp.float32),
            pltpu.SemaphoreType.DMA,
            pltpu.SemaphoreType.DMA,
        ],
        compiler_params=pltpu.CompilerParams(collective_id=0),
    )(x)


# device time: 6495 ns/iter; 1.3093x vs baseline; 1.0122x over previous
import jax
import jax.numpy as jnp
from jax import lax
from jax.experimental import pallas as pl
from jax.experimental.pallas import tpu as pltpu

K = 8
NEG_INF = float("-inf")


def kernel(x):
    m, n = x.shape

    def body(x_ref, out_ref, cand_ref, send_sem, recv_sem):
        my_x = lax.axis_index("x")
        my_y = lax.axis_index("y")
        peer = (my_x, 1 - my_y)

        q = n // 4
        a = x_ref[:, 0 * q:1 * q]
        b = x_ref[:, 1 * q:2 * q]
        c = x_ref[:, 2 * q:3 * q]
        d = x_ref[:, 3 * q:4 * q]

        def ce(u, v):
            return jnp.maximum(u, v), jnp.minimum(u, v)

        p0, p1 = ce(a, b)
        p2, p3 = ce(c, d)
        s0, t2 = ce(p0, p2)
        t1, s3 = ce(p1, p3)
        s1, s2 = ce(t1, t2)

        cols = []
        for _ in range(K):
            mx = jnp.max(s0, axis=1, keepdims=True)
            cols.append(mx)
            hit = s0 == mx
            s0 = jnp.where(hit, s1, s0)
            s1 = jnp.where(hit, s2, s1)
            s2 = jnp.where(hit, s3, s2)
            s3 = jnp.where(hit, NEG_INF, s3)
        local_top = jnp.concatenate(cols, axis=1)
        cand_ref[0, :, :] = local_top.T

        barrier_sem = pltpu.get_barrier_semaphore()
        pl.semaphore_signal(
            barrier_sem, inc=1, device_id=peer,
            device_id_type=pl.DeviceIdType.MESH,
        )
        pl.semaphore_wait(barrier_sem, 1)

        rdma = pltpu.make_async_remote_copy(
            src_ref=cand_ref.at[0],
            dst_ref=cand_ref.at[1],
            send_sem=send_sem,
            recv_sem=recv_sem,
            device_id=peer,
            device_id_type=pl.DeviceIdType.MESH,
        )
        rdma.start()
        rdma.wait()

        L = [
            jnp.maximum(cand_ref[0, i, :], cand_ref[1, K - 1 - i, :])
            for i in range(K)
        ]

        def merge_ce(i, j):
            L[i], L[j] = jnp.maximum(L[i], L[j]), jnp.minimum(L[i], L[j])

        for (i, j) in [(0, 4), (1, 5), (2, 6), (3, 7),
                       (0, 2), (1, 3), (4, 6), (5, 7),
                       (0, 1), (2, 3), (4, 5), (6, 7)]:
            merge_ce(i, j)
        out_ref[:, :] = jnp.stack(L, axis=0).T

    return pl.pallas_call(
        body,
        out_shape=jax.ShapeDtypeStruct((m, K), jnp.float32),
        in_specs=[pl.BlockSpec(memory_space=pltpu.VMEM)],
        out_specs=pl.BlockSpec(memory_space=pltpu.VMEM),
        scratch_shapes=[
            pltpu.VMEM((2, K, m), jnp.float32),
            pltpu.SemaphoreType.DMA,
            pltpu.SemaphoreType.DMA,
        ],
        compiler_params=pltpu.CompilerParams(collective_id=0),
    )(x)
